# baseline (speedup 1.0000x reference)
import functools

import jax
import jax.numpy as jnp
from jax import lax
from jax.experimental import pallas as pl
from jax.experimental.pallas import tpu as pltpu
from jax.experimental.pallas import tpu_sc as plsc

N = 10000
D = 128
K = 10
ALPHA = 0.1
NC = 2
NS = 16
NTILES = NC * NS
W = 128
NWIN = 80
E_PAD = NTILES * NWIN * W
NPAD = 10240
RPT = NPAD // NS
RB = 400
GRID = N // RB

_mesh = plsc.VectorSubcoreMesh(core_axis_name="c", subcore_axis_name="s")

_CHUNKS = ((0, 128), (128, 128), (256, 128), (384, 128), (512, 128))


def _fill_rows(buf, nrows, ncols, value):
    v = jnp.full((16,), value, jnp.float32)

    @pl.loop(0, nrows)
    def _(r):
        for j in range(ncols // 16):
            buf[r, pl.ds(j * 16, 16)] = v


@functools.partial(
    pl.kernel,
    out_type=jax.ShapeDtypeStruct((NC, NPAD, D), jnp.float32),
    mesh=_mesh,
    scratch_types=[
        pltpu.VMEM((NWIN, W), jnp.int32),
        pltpu.VMEM((W, D), jnp.float32),
        pltpu.VMEM_SHARED((NPAD, D), jnp.float32),
    ],
)
def _deg_sc(dst_hbm, deg_hbm, dst_all, ones_v, agg_sh):
    c = lax.axis_index("c")
    s = lax.axis_index("s")
    wid = c * NS + s
    row0 = s * RPT
    _fill_rows(ones_v, W, D, 0.0)
    for lo, sz in _CHUNKS:
        pltpu.sync_copy(ones_v.at[pl.ds(0, sz)], agg_sh.at[pl.ds(row0 + lo, sz)])
    _fill_rows(ones_v, W, D, 1.0)
    pltpu.sync_copy(dst_hbm.at[pl.ds(wid * NWIN, NWIN)], dst_all)
    plsc.subcore_barrier()

    @pl.loop(0, NWIN)
    def _(w):
        pltpu.sync_copy(ones_v, agg_sh.at[dst_all.at[w]], add=True)

    plsc.subcore_barrier()
    for lo, sz in _CHUNKS:
        pltpu.sync_copy(agg_sh.at[pl.ds(row0 + lo, sz)],
                        deg_hbm.at[c].at[pl.ds(row0 + lo, sz)])


@functools.partial(
    pl.kernel,
    out_type=jax.ShapeDtypeStruct((NC, NPAD, D), jnp.float32),
    mesh=_mesh,
    scratch_types=[
        pltpu.VMEM((NWIN // 2, W), jnp.int32),
        pltpu.VMEM((NWIN // 2, W), jnp.int32),
        pltpu.VMEM((W, D), jnp.float32),
        pltpu.VMEM((W, D), jnp.float32),
        pltpu.SemaphoreType.DMA,
        pltpu.SemaphoreType.DMA,
        pltpu.VMEM_SHARED((NPAD, D), jnp.float32),
    ],
)
def _prop_sc(y_hbm, src_hbm, dst_hbm, agg_hbm,
             src_all, dst_all, rows0, rows1, gsem0, gsem1, agg_sh):
    c = lax.axis_index("c")
    s = lax.axis_index("s")
    wid = c * NS + s
    row0 = s * RPT
    hnw = NWIN // 2
    _fill_rows(rows0, W, D, 0.0)
    for lo, sz in _CHUNKS:
        pltpu.async_copy(rows0.at[pl.ds(0, sz)], agg_sh.at[pl.ds(row0 + lo, sz)], gsem0)
    for lo, sz in _CHUNKS:
        pltpu.make_async_copy(rows0.at[pl.ds(0, sz)],
                              agg_sh.at[pl.ds(row0 + lo, sz)], gsem0).wait()
    plsc.subcore_barrier()

    for half in range(2):
        hbase = wid * NWIN + half * hnw
        pltpu.sync_copy(src_hbm.at[pl.ds(hbase, hnw)], src_all)
        pltpu.sync_copy(dst_hbm.at[pl.ds(hbase, hnw)], dst_all)
        def _start_gather(w, buf, sem):
            pltpu.async_copy(y_hbm.at[src_all.at[w, pl.ds(0, W // 2)]],
                             buf.at[pl.ds(0, W // 2)], sem)
            pltpu.async_copy(y_hbm.at[src_all.at[w, pl.ds(W // 2, W // 2)]],
                             buf.at[pl.ds(W // 2, W // 2)], sem)

        def _wait_gather(w, buf, sem):
            for lo in (0, W // 2):
                pltpu.make_async_copy(y_hbm.at[src_all.at[w, pl.ds(lo, W // 2)]],
                                      buf.at[pl.ds(lo, W // 2)], sem).wait()

        _start_gather(0, rows0, gsem0)

        @pl.loop(0, hnw, step=2)
        def _(w):
            _wait_gather(w, rows0, gsem0)
            _start_gather(w + 1, rows1, gsem1)
            pltpu.sync_copy(rows0, agg_sh.at[dst_all.at[w]], add=True)
            _wait_gather(w + 1, rows1, gsem1)

            @pl.when(w + 2 < hnw)
            def _():
                _start_gather(w + 2, rows0, gsem0)

            pltpu.sync_copy(rows1, agg_sh.at[dst_all.at[w + 1]], add=True)

    plsc.subcore_barrier()
    for lo, sz in _CHUNKS:
        pltpu.async_copy(agg_sh.at[pl.ds(row0 + lo, sz)],
                         agg_hbm.at[c].at[pl.ds(row0 + lo, sz)], gsem0)
    for lo, sz in _CHUNKS:
        pltpu.make_async_copy(agg_sh.at[pl.ds(row0 + lo, sz)],
                              agg_hbm.at[c].at[pl.ds(row0 + lo, sz)], gsem0).wait()


def _mlp_body(x_ref, w1, b1, w2, b2, w3, b3, h_ref):
    h = jnp.maximum(jnp.dot(x_ref[...], w1[...],
                            preferred_element_type=jnp.float32) + b1[...], 0.0)
    h = jnp.maximum(jnp.dot(h, w2[...],
                            preferred_element_type=jnp.float32) + b2[...], 0.0)
    h_ref[...] = jnp.dot(h, w3[...],
                         preferred_element_type=jnp.float32) + b3[...]


_row_spec = pl.BlockSpec((RB, D), lambda i: (i, 0))
_w_spec = pl.BlockSpec((D, D), lambda i: (0, 0))
_b_spec = pl.BlockSpec((1, D), lambda i: (0, 0))

_mlp = pl.pallas_call(
    _mlp_body,
    grid=(GRID,),
    in_specs=[_row_spec, _w_spec, _b_spec, _w_spec, _b_spec, _w_spec, _b_spec],
    out_specs=_row_spec,
    out_shape=jax.ShapeDtypeStruct((N, D), jnp.float32),
)

_agg_spec0 = pl.BlockSpec((1, RB, D), lambda i: (0, i, 0))
_agg_spec1 = pl.BlockSpec((1, RB, D), lambda i: (1, i, 0))


def _prescale_body(d0, d1, h_ref, dinv_ref, y_ref):
    deg = d0[0] + d1[0]
    dcol = deg[:, 0:1]
    dinv = jnp.where(dcol > 0, lax.rsqrt(jnp.maximum(dcol, 1.0)), 0.0)
    dinv_m = jnp.broadcast_to(dinv, (RB, D))
    dinv_ref[...] = dinv_m
    y_ref[...] = h_ref[...] * dinv_m


_prescale = pl.pallas_call(
    _prescale_body,
    grid=(GRID,),
    in_specs=[_agg_spec0, _agg_spec1, _row_spec],
    out_specs=[_row_spec, _row_spec],
    out_shape=[jax.ShapeDtypeStruct((N, D), jnp.float32),
               jax.ShapeDtypeStruct((N, D), jnp.float32)],
)


def _combine_body(a0, a1, dinv_ref, h_ref, y_ref):
    dm = dinv_ref[...]
    out = (1.0 - ALPHA) * dm * (a0[0] + a1[0]) + ALPHA * h_ref[...]
    y_ref[...] = dm * out


_combine = pl.pallas_call(
    _combine_body,
    grid=(GRID,),
    in_specs=[_agg_spec0, _agg_spec1, _row_spec, _row_spec],
    out_specs=_row_spec,
    out_shape=jax.ShapeDtypeStruct((N, D), jnp.float32),
)


def _final_body(a0, a1, dinv_ref, h_ref, o_ref):
    out = (1.0 - ALPHA) * dinv_ref[...] * (a0[0] + a1[0]) + ALPHA * h_ref[...]
    m = jnp.max(out, axis=1, keepdims=True)
    lse = m + jnp.log(jnp.sum(jnp.exp(out - m), axis=1, keepdims=True))
    o_ref[...] = out - lse


_final = pl.pallas_call(
    _final_body,
    grid=(GRID,),
    in_specs=[_agg_spec0, _agg_spec1, _row_spec, _row_spec],
    out_specs=_row_spec,
    out_shape=jax.ShapeDtypeStruct((N, D), jnp.float32),
)


def kernel(x, edge_index, W1, b1, W2, b2, W3, b3):
    src = edge_index[0].astype(jnp.int32)
    dst = edge_index[1].astype(jnp.int32)
    e = src.shape[0]
    npad = E_PAD - e
    ar = jnp.arange(npad, dtype=jnp.int32)
    src_p = jnp.concatenate([src, (ar * 97) % N]).reshape(NTILES * NWIN, W)
    dst_p = jnp.concatenate([dst, N + (ar % 16)]).reshape(NTILES * NWIN, W)

    h = _mlp(x, W1, b1.reshape(1, D), W2, b2.reshape(1, D), W3, b3.reshape(1, D))
    deg = _deg_sc(dst_p)
    dinv_m, y = _prescale(deg, deg, h)
    for k in range(K):
        agg = _prop_sc(y, src_p, dst_p)
        if k < K - 1:
            y = _combine(agg, agg, dinv_m, h)
        else:
            return _final(agg, agg, dinv_m, h)


# speedup vs baseline: 1.0046x; 1.0046x over previous
import functools

import jax
import jax.numpy as jnp
from jax import lax
from jax.experimental import pallas as pl
from jax.experimental.pallas import tpu as pltpu
from jax.experimental.pallas import tpu_sc as plsc

N = 10000
D = 128
K = 10
ALPHA = 0.1
NC = 2
NS = 16
NTILES = NC * NS
W = 128
NWIN = 80
E_PAD = NTILES * NWIN * W
NPAD = 10240
RPT = NPAD // NS
RB = 400
GRID = N // RB

_mesh = plsc.VectorSubcoreMesh(core_axis_name="c", subcore_axis_name="s")

_CHUNKS = ((0, 128), (128, 128), (256, 128), (384, 128), (512, 128))


def _fill_rows(buf, nrows, ncols, value):
    v = jnp.full((16,), value, jnp.float32)

    @pl.loop(0, nrows)
    def _(r):
        for j in range(ncols // 16):
            buf[r, pl.ds(j * 16, 16)] = v


@functools.partial(
    pl.kernel,
    out_type=jax.ShapeDtypeStruct((NC, NPAD, D), jnp.float32),
    mesh=_mesh,
    scratch_types=[
        pltpu.VMEM((NWIN, W), jnp.int32),
        pltpu.VMEM((W, D), jnp.float32),
        pltpu.VMEM_SHARED((NPAD, D), jnp.float32),
    ],
)
def _deg_sc(dst_hbm, deg_hbm, dst_all, ones_v, agg_sh):
    c = lax.axis_index("c")
    s = lax.axis_index("s")
    wid = c * NS + s
    row0 = s * RPT
    _fill_rows(ones_v, W, D, 0.0)
    for lo, sz in _CHUNKS:
        pltpu.sync_copy(ones_v.at[pl.ds(0, sz)], agg_sh.at[pl.ds(row0 + lo, sz)])
    _fill_rows(ones_v, W, D, 1.0)
    pltpu.sync_copy(dst_hbm.at[pl.ds(wid * NWIN, NWIN)], dst_all)
    plsc.subcore_barrier()

    @pl.loop(0, NWIN)
    def _(w):
        pltpu.sync_copy(ones_v, agg_sh.at[dst_all.at[w]], add=True)

    plsc.subcore_barrier()
    pltpu.sync_copy(agg_sh.at[pl.ds(row0, RPT)], deg_hbm.at[c].at[pl.ds(row0, RPT)])


@functools.partial(
    pl.kernel,
    out_type=jax.ShapeDtypeStruct((NC, NPAD, D), jnp.float32),
    mesh=_mesh,
    scratch_types=[
        pltpu.VMEM((NWIN // 2, W), jnp.int32),
        pltpu.VMEM((NWIN // 2, W), jnp.int32),
        pltpu.VMEM((W, D), jnp.float32),
        pltpu.VMEM((W, D), jnp.float32),
        pltpu.SemaphoreType.DMA,
        pltpu.SemaphoreType.DMA,
        pltpu.VMEM_SHARED((NPAD, D), jnp.float32),
    ],
)
def _prop_sc(y_hbm, src_hbm, dst_hbm, agg_hbm,
             src_all, dst_all, rows0, rows1, gsem0, gsem1, agg_sh):
    c = lax.axis_index("c")
    s = lax.axis_index("s")
    wid = c * NS + s
    row0 = s * RPT
    hnw = NWIN // 2
    _fill_rows(rows0, W, D, 0.0)
    for lo, sz in _CHUNKS:
        pltpu.async_copy(rows0.at[pl.ds(0, sz)], agg_sh.at[pl.ds(row0 + lo, sz)], gsem0)
    for lo, sz in _CHUNKS:
        pltpu.make_async_copy(rows0.at[pl.ds(0, sz)],
                              agg_sh.at[pl.ds(row0 + lo, sz)], gsem0).wait()
    plsc.subcore_barrier()

    for half in range(2):
        hbase = wid * NWIN + half * hnw
        pltpu.sync_copy(src_hbm.at[pl.ds(hbase, hnw)], src_all)
        pltpu.sync_copy(dst_hbm.at[pl.ds(hbase, hnw)], dst_all)
        pltpu.async_copy(y_hbm.at[src_all.at[0]], rows0, gsem0)

        @pl.loop(0, hnw, step=2)
        def _(w):
            pltpu.make_async_copy(y_hbm.at[src_all.at[w]], rows0, gsem0).wait()
            pltpu.async_copy(y_hbm.at[src_all.at[w + 1]], rows1, gsem1)
            pltpu.sync_copy(rows0, agg_sh.at[dst_all.at[w]], add=True)
            pltpu.make_async_copy(y_hbm.at[src_all.at[w + 1]], rows1, gsem1).wait()

            @pl.when(w + 2 < hnw)
            def _():
                pltpu.async_copy(y_hbm.at[src_all.at[w + 2]], rows0, gsem0)

            pltpu.sync_copy(rows1, agg_sh.at[dst_all.at[w + 1]], add=True)

    plsc.subcore_barrier()
    pltpu.sync_copy(agg_sh.at[pl.ds(row0, RPT)], agg_hbm.at[c].at[pl.ds(row0, RPT)])


def _mlp_body(x_ref, w1, b1, w2, b2, w3, b3, h_ref):
    h = jnp.maximum(jnp.dot(x_ref[...], w1[...],
                            preferred_element_type=jnp.float32) + b1[...], 0.0)
    h = jnp.maximum(jnp.dot(h, w2[...],
                            preferred_element_type=jnp.float32) + b2[...], 0.0)
    h_ref[...] = jnp.dot(h, w3[...],
                         preferred_element_type=jnp.float32) + b3[...]


_row_spec = pl.BlockSpec((RB, D), lambda i: (i, 0))
_w_spec = pl.BlockSpec((D, D), lambda i: (0, 0))
_b_spec = pl.BlockSpec((1, D), lambda i: (0, 0))

_mlp = pl.pallas_call(
    _mlp_body,
    grid=(GRID,),
    in_specs=[_row_spec, _w_spec, _b_spec, _w_spec, _b_spec, _w_spec, _b_spec],
    out_specs=_row_spec,
    out_shape=jax.ShapeDtypeStruct((N, D), jnp.float32),
)

_agg_spec0 = pl.BlockSpec((1, RB, D), lambda i: (0, i, 0))
_agg_spec1 = pl.BlockSpec((1, RB, D), lambda i: (1, i, 0))


def _prescale_body(d0, d1, h_ref, dinv_ref, y_ref):
    deg = d0[0] + d1[0]
    dcol = deg[:, 0:1]
    dinv = jnp.where(dcol > 0, lax.rsqrt(jnp.maximum(dcol, 1.0)), 0.0)
    dinv_m = jnp.broadcast_to(dinv, (RB, D))
    dinv_ref[...] = dinv_m
    y_ref[...] = h_ref[...] * dinv_m


_prescale = pl.pallas_call(
    _prescale_body,
    grid=(GRID,),
    in_specs=[_agg_spec0, _agg_spec1, _row_spec],
    out_specs=[_row_spec, _row_spec],
    out_shape=[jax.ShapeDtypeStruct((N, D), jnp.float32),
               jax.ShapeDtypeStruct((N, D), jnp.float32)],
)


def _combine_body(a0, a1, dinv_ref, h_ref, y_ref):
    dm = dinv_ref[...]
    out = (1.0 - ALPHA) * dm * (a0[0] + a1[0]) + ALPHA * h_ref[...]
    y_ref[...] = dm * out


_combine = pl.pallas_call(
    _combine_body,
    grid=(GRID,),
    in_specs=[_agg_spec0, _agg_spec1, _row_spec, _row_spec],
    out_specs=_row_spec,
    out_shape=jax.ShapeDtypeStruct((N, D), jnp.float32),
)


def _final_body(a0, a1, dinv_ref, h_ref, o_ref):
    out = (1.0 - ALPHA) * dinv_ref[...] * (a0[0] + a1[0]) + ALPHA * h_ref[...]
    m = jnp.max(out, axis=1, keepdims=True)
    lse = m + jnp.log(jnp.sum(jnp.exp(out - m), axis=1, keepdims=True))
    o_ref[...] = out - lse


_final = pl.pallas_call(
    _final_body,
    grid=(GRID,),
    in_specs=[_agg_spec0, _agg_spec1, _row_spec, _row_spec],
    out_specs=_row_spec,
    out_shape=jax.ShapeDtypeStruct((N, D), jnp.float32),
)


def kernel(x, edge_index, W1, b1, W2, b2, W3, b3):
    src = edge_index[0].astype(jnp.int32)
    dst = edge_index[1].astype(jnp.int32)
    e = src.shape[0]
    npad = E_PAD - e
    ar = jnp.arange(npad, dtype=jnp.int32)
    src_p = jnp.concatenate([src, (ar * 97) % N]).reshape(NTILES * NWIN, W)
    dst_p = jnp.concatenate([dst, N + (ar % 16)]).reshape(NTILES * NWIN, W)

    h = _mlp(x, W1, b1.reshape(1, D), W2, b2.reshape(1, D), W3, b3.reshape(1, D))
    deg = _deg_sc(dst_p)
    dinv_m, y = _prescale(deg, deg, h)
    for k in range(K):
        agg = _prop_sc(y, src_p, dst_p)
        if k < K - 1:
            y = _combine(agg, agg, dinv_m, h)
        else:
            return _final(agg, agg, dinv_m, h)
